# Initial kernel scaffold; baseline (speedup 1.0000x reference)
import jax
import jax.numpy as jnp
from jax.experimental import pallas as pl


def kernel(x, Wq, Wk, Wv, Wo, gq, gk):
    raise NotImplementedError("write your pallas kernel here")


# speedup vs baseline: 1.0454x; 1.0454x over previous
import jax
import jax.numpy as jnp
from jax.experimental import pallas as pl

DIM = 1024
HEADS = 16
HDIM = DIM // HEADS
SEQ = 2048
BH, BW = 8, 8
WH, WW = 6, 6
EPS = 1e-6
TPB = SEQ // (BH * BW)
ROW = BW * TPB
KVW = WH * ROW
SCALE = 1.0 / (HDIM ** 0.5)


def _mm_kernel(a_ref, b_ref, o_ref):
    o_ref[...] = jnp.dot(a_ref[...], b_ref[...],
                         preferred_element_type=jnp.float32)


HPG = 2


def _attn_kernel(q_ref, k_ref, v_ref, gq_ref, gk_ref, o_ref):
    r = pl.program_id(1)
    start = jnp.clip(r - WH // 2, 0, BH - WH) * ROW

    qi = jax.lax.broadcasted_iota(jnp.int32, (ROW, KVW), 0)
    kj = jax.lax.broadcasted_iota(jnp.int32, (ROW, KVW), 1)
    ci = qi // TPB
    rj = start // ROW + kj // ROW
    cj = (kj // TPB) % BW
    valid = ((rj >= r - WH // 2) & (rj <= r + WH - 1 - WH // 2) &
             (cj >= ci - WW // 2) & (cj <= ci + WW - 1 - WW // 2))

    for u in range(HPG):
        sl = slice(u * HDIM, (u + 1) * HDIM)
        q = q_ref[:, sl]
        q = q * jax.lax.rsqrt(jnp.mean(q * q, axis=-1, keepdims=True) + EPS)
        q = q * gq_ref[...]

        k = k_ref[pl.ds(start, KVW), sl]
        k = k * jax.lax.rsqrt(jnp.mean(k * k, axis=-1, keepdims=True) + EPS)
        k = k * gk_ref[...]
        v = v_ref[pl.ds(start, KVW), sl]

        s = jax.lax.dot_general(q, k, (((1,), (1,)), ((), ())),
                                preferred_element_type=jnp.float32) * SCALE
        s = jnp.where(valid, s, jnp.float32(-1e9))
        p = jax.nn.softmax(s, axis=-1)
        o_ref[:, sl] = jnp.dot(p, v, preferred_element_type=jnp.float32)


def kernel(x, Wq, Wk, Wv, Wo, gq, gk):
    B = x.shape[0]
    x2 = x.reshape(SEQ, DIM)
    Wqkv = jnp.concatenate([Wq, Wk, Wv], axis=1)
    gq2 = gq.reshape(1, HDIM)
    gk2 = gk.reshape(1, HDIM)

    qkv = pl.pallas_call(
        _mm_kernel,
        grid=(SEQ // ROW,),
        in_specs=[
            pl.BlockSpec((ROW, DIM), lambda i: (i, 0)),
            pl.BlockSpec((DIM, 3 * DIM), lambda i: (0, 0)),
        ],
        out_specs=pl.BlockSpec((ROW, 3 * DIM), lambda i: (i, 0)),
        out_shape=jax.ShapeDtypeStruct((SEQ, 3 * DIM), jnp.float32),
    )(x2, Wqkv)

    npairs = HEADS // HPG
    cw = HPG * HDIM
    attn = pl.pallas_call(
        _attn_kernel,
        grid=(npairs, BH),
        in_specs=[
            pl.BlockSpec((ROW, cw), lambda p, r: (r, p)),
            pl.BlockSpec((SEQ, cw), lambda p, r: (0, npairs + p)),
            pl.BlockSpec((SEQ, cw), lambda p, r: (0, 2 * npairs + p)),
            pl.BlockSpec((1, HDIM), lambda p, r: (0, 0)),
            pl.BlockSpec((1, HDIM), lambda p, r: (0, 0)),
        ],
        out_specs=pl.BlockSpec((ROW, cw), lambda p, r: (r, p)),
        out_shape=jax.ShapeDtypeStruct((SEQ, DIM), jnp.float32),
    )(qkv, qkv, qkv, gq2, gk2)

    out = pl.pallas_call(
        _mm_kernel,
        grid=(SEQ // ROW,),
        in_specs=[
            pl.BlockSpec((ROW, DIM), lambda i: (i, 0)),
            pl.BlockSpec((DIM, DIM), lambda i: (0, 0)),
        ],
        out_specs=pl.BlockSpec((ROW, DIM), lambda i: (i, 0)),
        out_shape=jax.ShapeDtypeStruct((SEQ, DIM), jnp.float32),
    )(attn, Wo)

    return out.reshape(B, SEQ, DIM)


# speedup vs baseline: 1.9922x; 1.9056x over previous
import jax
import jax.numpy as jnp
from jax.experimental import pallas as pl
from jax.experimental.pallas import tpu as pltpu

DIM = 1024
HEADS = 16
HDIM = DIM // HEADS
SEQ = 2048
BH, BW = 8, 8
WH, WW = 6, 6
EPS = 1e-6
TPB = SEQ // (BH * BW)
ROW = BW * TPB
KVW = WH * ROW
SCALE = 1.0 / (HDIM ** 0.5)
HPG = 2
NEG = -1e9


def _mm_kernel(a_ref, b_ref, o_ref):
    o_ref[...] = jnp.dot(a_ref[...], b_ref[...],
                         preferred_element_type=jnp.float32
                         ).astype(o_ref.dtype)


def _attn_kernel(q_ref, k_ref, v_ref, gq_ref, gk_ref, o_ref,
                 kn_ref, vx_ref, cb_ref):
    p_idx = pl.program_id(0)
    r = pl.program_id(1)

    @pl.when(jnp.logical_and(p_idx == 0, r == 0))
    def _():
        qi = jax.lax.broadcasted_iota(jnp.int32, (ROW, KVW), 0)
        kj = jax.lax.broadcasted_iota(jnp.int32, (ROW, KVW), 1)
        d = (kj // TPB) % BW - qi // TPB
        ok = (d >= -(WW // 2)) & (d <= WW - 1 - WW // 2)
        cb_ref[...] = jnp.where(ok, 0.0, NEG).astype(jnp.float32)

    @pl.when(r == 0)
    def _():
        k = k_ref[...].astype(jnp.float32)
        v = v_ref[...]
        for u in range(HPG):
            sl = slice(u * HDIM, (u + 1) * HDIM)
            ku = k[:, sl]
            ku = ku * jax.lax.rsqrt(jnp.mean(ku * ku, -1, keepdims=True) + EPS)
            kn_ref[:, sl] = (ku * gk_ref[...]).astype(jnp.bfloat16)
            vx_ref[:, 2 * u * HDIM:(2 * u + 1) * HDIM] = v[:, sl]
            vx_ref[:, (2 * u + 1) * HDIM:(2 * u + 2) * HDIM] = jnp.ones(
                (SEQ, HDIM), jnp.bfloat16)

    start = jnp.clip(r - WH // 2, 0, BH - WH) * ROW
    rj = start // ROW + jax.lax.broadcasted_iota(jnp.int32, (8, KVW), 1) // ROW
    rok = (rj >= r - WH // 2) & (rj <= r + WH - 1 - WH // 2)
    bias = cb_ref[...] + jnp.where(rok, 0.0, NEG).astype(jnp.float32)[:1, :]

    for u in range(HPG):
        sl = slice(u * HDIM, (u + 1) * HDIM)
        q = q_ref[:, sl].astype(jnp.float32)
        q = q * (jax.lax.rsqrt(jnp.mean(q * q, -1, keepdims=True) + EPS)
                 * SCALE)
        qb = (q * gq_ref[...]).astype(jnp.bfloat16)
        kn = kn_ref[pl.ds(start, KVW), sl]
        s = jax.lax.dot_general(qb, kn, (((1,), (1,)), ((), ())),
                                preferred_element_type=jnp.float32)
        e = jnp.exp(s + bias).astype(jnp.bfloat16)
        pv = jnp.dot(e, vx_ref[pl.ds(start, KVW), 2 * u * HDIM:
                               (2 * u + 2) * HDIM],
                     preferred_element_type=jnp.float32)
        o_ref[:, sl] = (pv[:, :HDIM] *
                        (1.0 / pv[:, HDIM:HDIM + 1])).astype(jnp.bfloat16)


def kernel(x, Wq, Wk, Wv, Wo, gq, gk):
    B = x.shape[0]
    x2 = x.reshape(SEQ, DIM).astype(jnp.bfloat16)
    Wqkv = jnp.concatenate([Wq, Wk, Wv], axis=1).astype(jnp.bfloat16)
    gq2 = gq.reshape(1, HDIM)
    gk2 = gk.reshape(1, HDIM)

    qkv = pl.pallas_call(
        _mm_kernel,
        grid=(SEQ // ROW,),
        in_specs=[
            pl.BlockSpec((ROW, DIM), lambda i: (i, 0)),
            pl.BlockSpec((DIM, 3 * DIM), lambda i: (0, 0)),
        ],
        out_specs=pl.BlockSpec((ROW, 3 * DIM), lambda i: (i, 0)),
        out_shape=jax.ShapeDtypeStruct((SEQ, 3 * DIM), jnp.bfloat16),
    )(x2, Wqkv)

    npairs = HEADS // HPG
    cw = HPG * HDIM
    attn = pl.pallas_call(
        _attn_kernel,
        grid=(npairs, BH),
        in_specs=[
            pl.BlockSpec((ROW, cw), lambda p, r: (r, p)),
            pl.BlockSpec((SEQ, cw), lambda p, r: (0, npairs + p)),
            pl.BlockSpec((SEQ, cw), lambda p, r: (0, 2 * npairs + p)),
            pl.BlockSpec((1, HDIM), lambda p, r: (0, 0)),
            pl.BlockSpec((1, HDIM), lambda p, r: (0, 0)),
        ],
        out_specs=pl.BlockSpec((ROW, cw), lambda p, r: (r, p)),
        out_shape=jax.ShapeDtypeStruct((SEQ, DIM), jnp.bfloat16),
        scratch_shapes=[
            pltpu.VMEM((SEQ, cw), jnp.bfloat16),
            pltpu.VMEM((SEQ, 2 * cw), jnp.bfloat16),
            pltpu.VMEM((ROW, KVW), jnp.float32),
        ],
    )(qkv, qkv, qkv, gq2, gk2)

    out = pl.pallas_call(
        _mm_kernel,
        grid=(SEQ // ROW,),
        in_specs=[
            pl.BlockSpec((ROW, DIM), lambda i: (i, 0)),
            pl.BlockSpec((DIM, DIM), lambda i: (0, 0)),
        ],
        out_specs=pl.BlockSpec((ROW, DIM), lambda i: (i, 0)),
        out_shape=jax.ShapeDtypeStruct((SEQ, DIM), jnp.float32),
    )(attn, Wo.astype(jnp.bfloat16))

    return out.reshape(B, SEQ, DIM)


# speedup vs baseline: 2.7490x; 1.3799x over previous
import jax
import jax.numpy as jnp
from jax.experimental import pallas as pl
from jax.experimental.pallas import tpu as pltpu

DIM = 1024
HEADS = 16
HDIM = DIM // HEADS
SEQ = 2048
BH, BW = 8, 8
WH, WW = 6, 6
EPS = 1e-6
TPB = SEQ // (BH * BW)
ROW = BW * TPB
KVW = WH * ROW
SCALE = 1.0 / (HDIM ** 0.5)
HPG = 2
CW = HPG * HDIM
NEG = -1e9


def _mm_kernel(a_ref, b_ref, o_ref):
    o_ref[...] = jnp.dot(a_ref[...], b_ref[...],
                         preferred_element_type=jnp.float32
                         ).astype(o_ref.dtype)


def _rowmeans(x2, n):
    i = jax.lax.broadcasted_iota(jnp.int32, (n, n), 0)
    j = jax.lax.broadcasted_iota(jnp.int32, (n, n), 1)
    ms = jnp.where((i // HDIM) == (j // HDIM), 1.0 / HDIM, 0.0
                   ).astype(jnp.bfloat16)
    return jnp.dot(x2.astype(jnp.bfloat16), ms,
                   preferred_element_type=jnp.float32)


def _attn_kernel(q_ref, k_ref, v_ref, gq_ref, gk_ref, o_ref,
                 kn_ref, vx_ref, cb_ref):
    qi = jax.lax.broadcasted_iota(jnp.int32, (ROW, KVW), 0)
    kj = jax.lax.broadcasted_iota(jnp.int32, (ROW, KVW), 1)
    d = (kj // TPB) % BW - qi // TPB
    ok = (d >= -(WW // 2)) & (d <= WW - 1 - WW // 2)
    cb_ref[...] = jnp.where(ok, 0.0, NEG).astype(jnp.float32)

    k = k_ref[...].astype(jnp.float32)
    km = _rowmeans(k * k, CW)
    kn_ref[...] = (k * jax.lax.rsqrt(km + EPS) * gk_ref[...]
                   ).astype(jnp.bfloat16)
    v = v_ref[...]
    one = jnp.ones((SEQ, HDIM), jnp.bfloat16)
    for u in range(HPG):
        vx_ref[:, 2 * u * HDIM:(2 * u + 1) * HDIM] = \
            v[:, u * HDIM:(u + 1) * HDIM]
        vx_ref[:, (2 * u + 1) * HDIM:(2 * u + 2) * HDIM] = one

    for r in range(BH):
        lo = max(r - WH // 2, 0) * ROW
        hi = min(r + WH - WH // 2, BH) * ROW
        w = hi - lo

        q = q_ref[r * ROW:(r + 1) * ROW, :].astype(jnp.float32)
        qm = _rowmeans(q * q, CW)
        qn = (q * jax.lax.rsqrt(qm + EPS) * gq_ref[...]).astype(jnp.bfloat16)

        for u in range(HPG):
            s = jax.lax.dot_general(
                qn[:, u * HDIM:(u + 1) * HDIM], kn_ref[lo:hi,
                                                       u * HDIM:(u + 1) * HDIM],
                (((1,), (1,)), ((), ())),
                preferred_element_type=jnp.float32)
            e = jnp.exp(s + cb_ref[:, :w]).astype(jnp.bfloat16)
            pv = jnp.dot(e, vx_ref[lo:hi, 2 * u * HDIM:(2 * u + 2) * HDIM],
                         preferred_element_type=jnp.float32)
            o_ref[r * ROW:(r + 1) * ROW, u * HDIM:(u + 1) * HDIM] = (
                pv[:, :HDIM] * (1.0 / pv[:, HDIM:HDIM + 1])
            ).astype(jnp.bfloat16)


def kernel(x, Wq, Wk, Wv, Wo, gq, gk):
    B = x.shape[0]
    x2 = x.reshape(SEQ, DIM).astype(jnp.bfloat16)
    Wqkv = jnp.concatenate([Wq, Wk, Wv], axis=1).astype(jnp.bfloat16)
    gq2 = jnp.tile(gq * SCALE, HPG).reshape(1, CW)
    gk2 = jnp.tile(gk, HPG).reshape(1, CW)

    qkv = pl.pallas_call(
        _mm_kernel,
        grid=(SEQ // ROW,),
        in_specs=[
            pl.BlockSpec((ROW, DIM), lambda i: (i, 0)),
            pl.BlockSpec((DIM, 3 * DIM), lambda i: (0, 0)),
        ],
        out_specs=pl.BlockSpec((ROW, 3 * DIM), lambda i: (i, 0)),
        out_shape=jax.ShapeDtypeStruct((SEQ, 3 * DIM), jnp.bfloat16),
        compiler_params=pltpu.CompilerParams(
            dimension_semantics=("parallel",)),
    )(x2, Wqkv)

    npairs = HEADS // HPG
    attn = pl.pallas_call(
        _attn_kernel,
        grid=(npairs,),
        in_specs=[
            pl.BlockSpec((SEQ, CW), lambda p: (0, p)),
            pl.BlockSpec((SEQ, CW), lambda p: (0, npairs + p)),
            pl.BlockSpec((SEQ, CW), lambda p: (0, 2 * npairs + p)),
            pl.BlockSpec((1, CW), lambda p: (0, 0)),
            pl.BlockSpec((1, CW), lambda p: (0, 0)),
        ],
        out_specs=pl.BlockSpec((SEQ, CW), lambda p: (0, p)),
        out_shape=jax.ShapeDtypeStruct((SEQ, DIM), jnp.bfloat16),
        scratch_shapes=[
            pltpu.VMEM((SEQ, CW), jnp.bfloat16),
            pltpu.VMEM((SEQ, 2 * CW), jnp.bfloat16),
            pltpu.VMEM((ROW, KVW), jnp.float32),
        ],
        compiler_params=pltpu.CompilerParams(
            dimension_semantics=("parallel",)),
    )(qkv, qkv, qkv, gq2, gk2)

    out = pl.pallas_call(
        _mm_kernel,
        grid=(SEQ // ROW,),
        in_specs=[
            pl.BlockSpec((ROW, DIM), lambda i: (i, 0)),
            pl.BlockSpec((DIM, DIM), lambda i: (0, 0)),
        ],
        out_specs=pl.BlockSpec((ROW, DIM), lambda i: (i, 0)),
        out_shape=jax.ShapeDtypeStruct((SEQ, DIM), jnp.float32),
        compiler_params=pltpu.CompilerParams(
            dimension_semantics=("parallel",)),
    )(attn, Wo.astype(jnp.bfloat16))

    return out.reshape(B, SEQ, DIM)
